# baseline (speedup 1.0000x reference)
import functools

import jax
import jax.numpy as jnp
from jax import lax
from jax.experimental import pallas as pl
from jax.experimental.pallas import tpu as pltpu
from jax.experimental.pallas import tpu_sc as plsc

F32 = jnp.float32

NC = 2
NS = 16
LANE = 128


def _mesh():
    return plsc.VectorSubcoreMesh(
        core_axis_name="c", subcore_axis_name="s", num_cores=NC, num_subcores=NS
    )


def _make_deg(n_pad, e_rows):
    rows_per_tile = e_rows // (NC * NS)
    n_per_tile = n_pad // NS

    @functools.partial(
        pl.kernel,
        out_type=(
            jax.ShapeDtypeStruct((n_pad,), F32),
            jax.ShapeDtypeStruct((n_pad,), F32),
        ),
        mesh=_mesh(),
        scratch_types=[
            pltpu.VMEM_SHARED((n_pad,), F32),
            pltpu.VMEM((rows_per_tile, LANE), jnp.int32),
            pltpu.VMEM((LANE,), F32),
            pltpu.VMEM((n_per_tile,), F32),
        ],
    )
    def deg_kernel(dst2d, zeros1, out0, out1, acc, idx_v, ones_v, wb_v):
        c = lax.axis_index("c")
        s = lax.axis_index("s")
        pltpu.sync_copy(
            zeros1.at[pl.ds(s * n_per_tile, n_per_tile)],
            acc.at[pl.ds(s * n_per_tile, n_per_tile)],
        )
        for i in range(LANE // 16):
            ones_v[pl.ds(i * 16, 16)] = jnp.full((16,), 1.0, F32)
        row0 = c * (e_rows // NC) + s * rows_per_tile
        pltpu.sync_copy(dst2d.at[pl.ds(row0, rows_per_tile)], idx_v)
        plsc.subcore_barrier()

        def body(j, _):
            pltpu.sync_copy(ones_v, acc.at[idx_v.at[j]], add=True)
            return 0

        lax.fori_loop(0, rows_per_tile, body, 0)
        plsc.subcore_barrier()
        sl = pl.ds(s * n_per_tile, n_per_tile)
        pltpu.sync_copy(acc.at[sl], wb_v)

        @pl.when(c == 0)
        def _():
            pltpu.sync_copy(wb_v, out0.at[sl])

        @pl.when(c == 1)
        def _():
            pltpu.sync_copy(wb_v, out1.at[sl])

    return deg_kernel


AGG_DT = jnp.float32


CHK = 80
NR = 4


def _make_agg(n_pad, e_rows, half):
    rows_per_tile = e_rows // NS
    n = rows_per_tile
    n_per_tile = n_pad // NS
    wb_chunks = n_per_tile // CHK
    assert n % NR == 0 and n_per_tile % CHK == 0

    @functools.partial(
        pl.kernel,
        out_type=(
            jax.ShapeDtypeStruct((n_pad, half), F32),
            jax.ShapeDtypeStruct((n_pad, half), F32),
        ),
        mesh=_mesh(),
        scratch_types=[
            pltpu.VMEM_SHARED((n_pad, half), F32),
            [pltpu.VMEM((CHK,), jnp.int32) for _ in range(NR)],
            [pltpu.VMEM((CHK,), jnp.int32) for _ in range(NR)],
            [pltpu.VMEM((CHK, half), F32) for _ in range(NR)],
            [pltpu.SemaphoreType.DMA for _ in range(NR)],
            [pltpu.SemaphoreType.DMA for _ in range(NR)],
            [pltpu.SemaphoreType.DMA for _ in range(NR)],
            [pltpu.SemaphoreType.DMA for _ in range(NR)],
        ],
    )
    def agg_kernel(src2d, dst2d, tab_lo, tab_hi, zeros2,
                   out_lo, out_hi, acc, sidx, didx, rows_v,
                   s_sems, d_sems, g_sems, sc_sems):
        c = lax.axis_index("c")
        s = lax.axis_index("s")
        nsl = pl.ds(s * n_per_tile, n_per_tile)
        row0 = s * rows_per_tile

        def run(tab, out):
            def pf_src(j, u):
                pltpu.async_copy(src2d.at[row0 + j], sidx[u], s_sems[u])

            def pf_dst(j, u):
                pltpu.async_copy(dst2d.at[row0 + j], didx[u], d_sems[u])

            def wait_src(u):
                pltpu.make_async_copy(src2d.at[row0], sidx[u],
                                      s_sems[u]).wait()

            def wait_dst(u):
                pltpu.make_async_copy(dst2d.at[row0], didx[u],
                                      d_sems[u]).wait()

            def gather(u):
                pltpu.async_copy(tab.at[sidx[u]], rows_v[u], g_sems[u])

            def wait_gather(u):
                pltpu.make_async_copy(tab.at[sidx[0]], rows_v[u],
                                      g_sems[u]).wait()

            def scatter(u):
                pltpu.async_copy(rows_v[u], acc.at[didx[u]], sc_sems[u],
                                 add=True)

            def wait_scatter(u):
                pltpu.make_async_copy(rows_v[u], acc.at[didx[0]],
                                      sc_sems[u]).wait()

            for u in range(NR):
                pf_src(u, u)
            for u in range(2):
                pf_dst(u, u)
            pltpu.sync_copy(zeros2.at[nsl], acc.at[nsl])
            for u in range(2):
                wait_src(u)
                gather(u)
            plsc.subcore_barrier()

            def outer(i, _):
                for u in range(NR):
                    j = i * NR + u
                    u2 = (u + 2) % NR
                    wait_gather(u)
                    pl.when(j + NR < n)(lambda j=j, u=u: pf_src(j + NR, u))
                    wait_dst(u)
                    scatter(u)

                    def nxt(j=j, u=u, u2=u2):
                        wait_src(u2)
                        pl.when(j >= 2)(lambda u2=u2: wait_scatter(u2))
                        gather(u2)
                        pf_dst(j + 2, u2)

                    pl.when(j + 2 < n)(nxt)
                return 0

            lax.fori_loop(0, n // NR, outer, 0)
            for u in range(NR):
                wait_scatter(u)
            plsc.subcore_barrier()
            for q in range(wb_chunks):
                sl = pl.ds(s * n_per_tile + q * CHK, CHK)
                pltpu.sync_copy(acc.at[sl], rows_v[q % 2])
                pltpu.sync_copy(rows_v[q % 2], out.at[sl])

        @pl.when(c == 0)
        def _():
            run(tab_lo, out_lo)

        @pl.when(c == 1)
        def _():
            run(tab_hi, out_hi)

    return agg_kernel


def _tc_scale(deg0, deg1, x_pad, half):
    n_pad, fin = x_pad.shape
    blk = 1024
    grid = (n_pad // blk,)

    def body(d0, d1, x, lo, hi):
        dinv = lax.rsqrt(d0[...] + d1[...] + 1.0)
        xs = (x[...] * dinv[:, None]).astype(AGG_DT)
        lo[...] = xs[:, :half]
        hi[...] = xs[:, half:]

    return pl.pallas_call(
        body,
        grid=grid,
        in_specs=[
            pl.BlockSpec((blk,), lambda i: (i,)),
            pl.BlockSpec((blk,), lambda i: (i,)),
            pl.BlockSpec((blk, fin), lambda i: (i, 0)),
        ],
        out_specs=[
            pl.BlockSpec((blk, half), lambda i: (i, 0)),
            pl.BlockSpec((blk, half), lambda i: (i, 0)),
        ],
        out_shape=[
            jax.ShapeDtypeStruct((n_pad, half), AGG_DT),
            jax.ShapeDtypeStruct((n_pad, half), AGG_DT),
        ],
    )(deg0, deg1, x_pad)


def _tc_layer1(deg0, deg1, x_pad, u_lo, u_hi, W1, b1, W2, half):
    n_pad, fin = x_pad.shape
    fmid = W1.shape[1]
    blk = 1024
    grid = (n_pad // blk,)

    def body(d0, d1, x, ulo, uhi, w1, bb1, w2, t_out, tslo, tshi):
        dinv = lax.rsqrt(d0[...] + d1[...] + 1.0)
        u = jnp.concatenate([ulo[...], uhi[...]], axis=1).astype(F32)
        z = u * dinv[:, None] + x[...] * (dinv * dinv)[:, None]
        bf = jnp.bfloat16
        h = jnp.maximum(
            jnp.dot(z.astype(bf), w1[...].astype(bf),
                    preferred_element_type=F32) + bb1[...][None, :],
            0.0,
        )
        t = jnp.dot(h.astype(bf), w2[...].astype(bf),
                    preferred_element_type=F32)
        t_out[...] = t
        ts = (t * dinv[:, None]).astype(AGG_DT)
        tslo[...] = ts[:, :half]
        tshi[...] = ts[:, half:]

    return pl.pallas_call(
        body,
        grid=grid,
        in_specs=[
            pl.BlockSpec((blk,), lambda i: (i,)),
            pl.BlockSpec((blk,), lambda i: (i,)),
            pl.BlockSpec((blk, fin), lambda i: (i, 0)),
            pl.BlockSpec((blk, half), lambda i: (i, 0)),
            pl.BlockSpec((blk, half), lambda i: (i, 0)),
            pl.BlockSpec((fin, fmid), lambda i: (0, 0)),
            pl.BlockSpec((fmid,), lambda i: (0,)),
            pl.BlockSpec((fmid, fin), lambda i: (0, 0)),
        ],
        out_specs=[
            pl.BlockSpec((blk, fin), lambda i: (i, 0)),
            pl.BlockSpec((blk, half), lambda i: (i, 0)),
            pl.BlockSpec((blk, half), lambda i: (i, 0)),
        ],
        out_shape=[
            jax.ShapeDtypeStruct((n_pad, fin), F32),
            jax.ShapeDtypeStruct((n_pad, half), AGG_DT),
            jax.ShapeDtypeStruct((n_pad, half), AGG_DT),
        ],
    )(deg0, deg1, x_pad, u_lo, u_hi, W1, b1, W2)


def _tc_finish(deg0, deg1, t, u_lo, u_hi, b2):
    n_pad, fout = t.shape
    half = fout // 2
    blk = 1024
    grid = (n_pad // blk,)

    def body(d0, d1, tt, ulo, uhi, bb2, out):
        dinv = lax.rsqrt(d0[...] + d1[...] + 1.0)
        u = jnp.concatenate([ulo[...], uhi[...]], axis=1).astype(F32)
        z = u * dinv[:, None] + tt[...] * (dinv * dinv)[:, None] + bb2[...][None, :]
        r = jnp.maximum(z, 0.0)
        m = jnp.max(r, axis=1, keepdims=True)
        lse = m + jnp.log(jnp.sum(jnp.exp(r - m), axis=1, keepdims=True))
        out[...] = r - lse

    return pl.pallas_call(
        body,
        grid=grid,
        in_specs=[
            pl.BlockSpec((blk,), lambda i: (i,)),
            pl.BlockSpec((blk,), lambda i: (i,)),
            pl.BlockSpec((blk, fout), lambda i: (i, 0)),
            pl.BlockSpec((blk, half), lambda i: (i, 0)),
            pl.BlockSpec((blk, half), lambda i: (i, 0)),
            pl.BlockSpec((fout,), lambda i: (0,)),
        ],
        out_specs=pl.BlockSpec((blk, fout), lambda i: (i, 0)),
        out_shape=jax.ShapeDtypeStruct((n_pad, fout), F32),
    )(deg0, deg1, t, u_lo, u_hi, b2)


def kernel(x, edge_index, W1, b1, W2, b2):
    n, fin = x.shape
    half = fin // 2
    e = edge_index.shape[1]

    n_pad = ((n + 1 + 1023) // 1024) * 1024
    e_pad = ((e + NC * NS * LANE - 1) // (NC * NS * LANE)) * (NC * NS * LANE)

    ei = edge_index.astype(jnp.int32)
    pad = jnp.full((e_pad - e,), n, jnp.int32)
    src2d = jnp.concatenate([ei[0], pad]).reshape(e_pad // LANE, LANE)
    dst2d = jnp.concatenate([ei[1], pad]).reshape(e_pad // LANE, LANE)
    src2d80 = src2d.reshape(e_pad // CHK, CHK)
    dst2d80 = dst2d.reshape(e_pad // CHK, CHK)
    x_pad = jnp.pad(x, ((0, n_pad - n), (0, 0)))
    zeros1 = jnp.zeros((n_pad,), F32)
    zeros2 = jnp.zeros((n_pad, half), F32)

    deg0, deg1 = _make_deg(n_pad, e_pad // LANE)(dst2d, zeros1)
    xs_lo, xs_hi = _tc_scale(deg0, deg1, x_pad, half)
    agg = _make_agg(n_pad, e_pad // CHK, half)
    u1_lo, u1_hi = agg(src2d80, dst2d80, xs_lo, xs_hi, zeros2)
    t, ts_lo, ts_hi = _tc_layer1(deg0, deg1, x_pad, u1_lo, u1_hi, W1, b1, W2, half)
    u2_lo, u2_hi = agg(src2d80, dst2d80, ts_lo, ts_hi, zeros2)
    o = _tc_finish(deg0, deg1, t, u2_lo, u2_hi, b2)
    return o[:n]


# speedup vs baseline: 1.0533x; 1.0533x over previous
import functools

import jax
import jax.numpy as jnp
from jax import lax
from jax.experimental import pallas as pl
from jax.experimental.pallas import tpu as pltpu
from jax.experimental.pallas import tpu_sc as plsc

F32 = jnp.float32

NC = 2
NS = 16
LANE = 128


def _mesh():
    return plsc.VectorSubcoreMesh(
        core_axis_name="c", subcore_axis_name="s", num_cores=NC, num_subcores=NS
    )


def _make_deg(n_pad, e_rows):
    rows_per_tile = e_rows // (NC * NS)
    n_per_tile = n_pad // NS

    @functools.partial(
        pl.kernel,
        out_type=(
            jax.ShapeDtypeStruct((n_pad,), F32),
            jax.ShapeDtypeStruct((n_pad,), F32),
        ),
        mesh=_mesh(),
        scratch_types=[
            pltpu.VMEM_SHARED((n_pad,), F32),
            pltpu.VMEM((rows_per_tile, LANE), jnp.int32),
            pltpu.VMEM((LANE,), F32),
            pltpu.VMEM((n_per_tile,), F32),
        ],
    )
    def deg_kernel(dst2d, zeros1, out0, out1, acc, idx_v, ones_v, wb_v):
        c = lax.axis_index("c")
        s = lax.axis_index("s")
        pltpu.sync_copy(
            zeros1.at[pl.ds(s * n_per_tile, n_per_tile)],
            acc.at[pl.ds(s * n_per_tile, n_per_tile)],
        )
        for i in range(LANE // 16):
            ones_v[pl.ds(i * 16, 16)] = jnp.full((16,), 1.0, F32)
        row0 = c * (e_rows // NC) + s * rows_per_tile
        pltpu.sync_copy(dst2d.at[pl.ds(row0, rows_per_tile)], idx_v)
        plsc.subcore_barrier()

        def body(j, _):
            pltpu.sync_copy(ones_v, acc.at[idx_v.at[j]], add=True)
            return 0

        lax.fori_loop(0, rows_per_tile, body, 0)
        plsc.subcore_barrier()
        sl = pl.ds(s * n_per_tile, n_per_tile)
        pltpu.sync_copy(acc.at[sl], wb_v)

        @pl.when(c == 0)
        def _():
            pltpu.sync_copy(wb_v, out0.at[sl])

        @pl.when(c == 1)
        def _():
            pltpu.sync_copy(wb_v, out1.at[sl])

    return deg_kernel


AGG_DT = jnp.float32


def _make_agg(n_pad, e_rows, half):
    rows_per_tile = e_rows // NS
    n_per_tile = n_pad // NS
    wb_chunks = n_per_tile // LANE

    nib = 4
    ngb = 2
    assert rows_per_tile % nib == 0

    @functools.partial(
        pl.kernel,
        out_type=(
            jax.ShapeDtypeStruct((n_pad, half), AGG_DT),
            jax.ShapeDtypeStruct((n_pad, half), AGG_DT),
        ),
        mesh=_mesh(),
        scratch_types=[
            pltpu.VMEM_SHARED((n_pad, half), AGG_DT),
            [pltpu.VMEM((2, LANE), jnp.int32) for _ in range(nib)],
            [pltpu.VMEM((LANE, half), AGG_DT) for _ in range(ngb)],
            [pltpu.SemaphoreType.DMA for _ in range(nib)],
            [pltpu.SemaphoreType.DMA for _ in range(ngb)],
            [pltpu.SemaphoreType.DMA for _ in range(2)],
        ],
    )
    def agg_kernel(edg3d, tab_lo, tab_hi, zeros2,
                   out_lo, out_hi, acc, idx_v, rows_v, isems, gsems, wsems):
        c = lax.axis_index("c")
        s = lax.axis_index("s")
        nsl = pl.ds(s * n_per_tile, n_per_tile)
        row0 = s * rows_per_tile

        def run(tab, out):
            def prefetch(j, ib):
                pltpu.async_copy(edg3d.at[row0 + j], idx_v[ib], isems[ib])

            def wait_idx(ib):
                pltpu.make_async_copy(edg3d.at[row0], idx_v[ib],
                                      isems[ib]).wait()

            def gather(ib, gb):
                pltpu.async_copy(tab.at[idx_v[ib].at[0]], rows_v[gb],
                                 gsems[gb])

            def wait_gather(gb):
                pltpu.make_async_copy(tab.at[idx_v[0].at[0]], rows_v[gb],
                                      gsems[gb]).wait()

            for j in range(nib):
                prefetch(j, j)
            pltpu.sync_copy(zeros2.at[nsl], acc.at[nsl])
            for j in range(ngb):
                wait_idx(j)
                gather(j, j)
            plsc.subcore_barrier()

            def outer(i, _):
                for b in range(nib):
                    j = i * nib + b
                    gb = b % ngb
                    ib2 = (b + ngb) % nib
                    wait_gather(gb)
                    pltpu.sync_copy(rows_v[gb], acc.at[idx_v[b].at[1]],
                                    add=True)
                    pl.when(j + nib < rows_per_tile)(
                        lambda j=j, b=b: prefetch(j + nib, b))

                    def nxt_gather(ib2=ib2, gb=gb):
                        wait_idx(ib2)
                        gather(ib2, gb)

                    pl.when(j + ngb < rows_per_tile)(nxt_gather)
                return 0

            lax.fori_loop(0, rows_per_tile // nib, outer, 0)
            plsc.subcore_barrier()
            for q in range(wb_chunks):
                u = q % 2
                sl = pl.ds(s * n_per_tile + q * LANE, LANE)
                if q >= 2:
                    pltpu.make_async_copy(rows_v[u], out.at[sl],
                                          wsems[u]).wait()
                pltpu.sync_copy(acc.at[sl], rows_v[u])
                pltpu.async_copy(rows_v[u], out.at[sl], wsems[u])
            last = pl.ds(s * n_per_tile, LANE)
            for u in range(2):
                pltpu.make_async_copy(rows_v[u], out.at[last],
                                      wsems[u]).wait()

        @pl.when(c == 0)
        def _():
            run(tab_lo, out_lo)

        @pl.when(c == 1)
        def _():
            run(tab_hi, out_hi)

    return agg_kernel


def _tc_scale(deg0, deg1, x_pad, half):
    n_pad, fin = x_pad.shape
    blk = 1024
    grid = (n_pad // blk,)

    def body(d0, d1, x, lo, hi):
        dinv = lax.rsqrt(d0[...] + d1[...] + 1.0)
        xs = (x[...] * dinv[:, None]).astype(AGG_DT)
        lo[...] = xs[:, :half]
        hi[...] = xs[:, half:]

    return pl.pallas_call(
        body,
        grid=grid,
        in_specs=[
            pl.BlockSpec((blk,), lambda i: (i,)),
            pl.BlockSpec((blk,), lambda i: (i,)),
            pl.BlockSpec((blk, fin), lambda i: (i, 0)),
        ],
        out_specs=[
            pl.BlockSpec((blk, half), lambda i: (i, 0)),
            pl.BlockSpec((blk, half), lambda i: (i, 0)),
        ],
        out_shape=[
            jax.ShapeDtypeStruct((n_pad, half), AGG_DT),
            jax.ShapeDtypeStruct((n_pad, half), AGG_DT),
        ],
    )(deg0, deg1, x_pad)


def _tc_layer1(deg0, deg1, x_pad, u_lo, u_hi, W1, b1, W2, half):
    n_pad, fin = x_pad.shape
    fmid = W1.shape[1]
    blk = 1024
    grid = (n_pad // blk,)

    def body(d0, d1, x, ulo, uhi, w1, bb1, w2, t_out, tslo, tshi):
        dinv = lax.rsqrt(d0[...] + d1[...] + 1.0)
        u = jnp.concatenate([ulo[...], uhi[...]], axis=1).astype(F32)
        z = u * dinv[:, None] + x[...] * (dinv * dinv)[:, None]
        bf = jnp.bfloat16
        h = jnp.maximum(
            jnp.dot(z.astype(bf), w1[...].astype(bf),
                    preferred_element_type=F32) + bb1[...][None, :],
            0.0,
        )
        t = jnp.dot(h.astype(bf), w2[...].astype(bf),
                    preferred_element_type=F32)
        t_out[...] = t
        ts = (t * dinv[:, None]).astype(AGG_DT)
        tslo[...] = ts[:, :half]
        tshi[...] = ts[:, half:]

    return pl.pallas_call(
        body,
        grid=grid,
        in_specs=[
            pl.BlockSpec((blk,), lambda i: (i,)),
            pl.BlockSpec((blk,), lambda i: (i,)),
            pl.BlockSpec((blk, fin), lambda i: (i, 0)),
            pl.BlockSpec((blk, half), lambda i: (i, 0)),
            pl.BlockSpec((blk, half), lambda i: (i, 0)),
            pl.BlockSpec((fin, fmid), lambda i: (0, 0)),
            pl.BlockSpec((fmid,), lambda i: (0,)),
            pl.BlockSpec((fmid, fin), lambda i: (0, 0)),
        ],
        out_specs=[
            pl.BlockSpec((blk, fin), lambda i: (i, 0)),
            pl.BlockSpec((blk, half), lambda i: (i, 0)),
            pl.BlockSpec((blk, half), lambda i: (i, 0)),
        ],
        out_shape=[
            jax.ShapeDtypeStruct((n_pad, fin), F32),
            jax.ShapeDtypeStruct((n_pad, half), AGG_DT),
            jax.ShapeDtypeStruct((n_pad, half), AGG_DT),
        ],
    )(deg0, deg1, x_pad, u_lo, u_hi, W1, b1, W2)


def _tc_finish(deg0, deg1, t, u_lo, u_hi, b2):
    n_pad, fout = t.shape
    half = fout // 2
    blk = 1024
    grid = (n_pad // blk,)

    def body(d0, d1, tt, ulo, uhi, bb2, out):
        dinv = lax.rsqrt(d0[...] + d1[...] + 1.0)
        u = jnp.concatenate([ulo[...], uhi[...]], axis=1).astype(F32)
        z = u * dinv[:, None] + tt[...] * (dinv * dinv)[:, None] + bb2[...][None, :]
        r = jnp.maximum(z, 0.0)
        m = jnp.max(r, axis=1, keepdims=True)
        lse = m + jnp.log(jnp.sum(jnp.exp(r - m), axis=1, keepdims=True))
        out[...] = r - lse

    return pl.pallas_call(
        body,
        grid=grid,
        in_specs=[
            pl.BlockSpec((blk,), lambda i: (i,)),
            pl.BlockSpec((blk,), lambda i: (i,)),
            pl.BlockSpec((blk, fout), lambda i: (i, 0)),
            pl.BlockSpec((blk, half), lambda i: (i, 0)),
            pl.BlockSpec((blk, half), lambda i: (i, 0)),
            pl.BlockSpec((fout,), lambda i: (0,)),
        ],
        out_specs=pl.BlockSpec((blk, fout), lambda i: (i, 0)),
        out_shape=jax.ShapeDtypeStruct((n_pad, fout), F32),
    )(deg0, deg1, t, u_lo, u_hi, b2)


def kernel(x, edge_index, W1, b1, W2, b2):
    n, fin = x.shape
    half = fin // 2
    e = edge_index.shape[1]

    n_pad = ((n + 1 + 1023) // 1024) * 1024
    e_pad = ((e + NC * NS * LANE - 1) // (NC * NS * LANE)) * (NC * NS * LANE)

    ei = edge_index.astype(jnp.int32)
    pad = jnp.full((e_pad - e,), n, jnp.int32)
    src2d = jnp.concatenate([ei[0], pad]).reshape(e_pad // LANE, LANE)
    dst2d = jnp.concatenate([ei[1], pad]).reshape(e_pad // LANE, LANE)
    edg3d = jnp.stack([src2d, dst2d], axis=1)
    x_pad = jnp.pad(x, ((0, n_pad - n), (0, 0)))
    zeros1 = jnp.zeros((n_pad,), F32)
    zeros2 = jnp.zeros((n_pad, half), AGG_DT)

    deg0, deg1 = _make_deg(n_pad, e_pad // LANE)(dst2d, zeros1)
    xs_lo, xs_hi = _tc_scale(deg0, deg1, x_pad, half)
    agg = _make_agg(n_pad, e_pad // LANE, half)
    u1_lo, u1_hi = agg(edg3d, xs_lo, xs_hi, zeros2)
    t, ts_lo, ts_hi = _tc_layer1(deg0, deg1, x_pad, u1_lo, u1_hi, W1, b1, W2, half)
    u2_lo, u2_hi = agg(edg3d, ts_lo, ts_hi, zeros2)
    o = _tc_finish(deg0, deg1, t, u2_lo, u2_hi, b2)
    return o[:n]


# speedup vs baseline: 2.4140x; 2.2919x over previous
import functools

import jax
import jax.numpy as jnp
from jax import lax
from jax.experimental import pallas as pl
from jax.experimental.pallas import tpu as pltpu
from jax.experimental.pallas import tpu_sc as plsc

F32 = jnp.float32

NC = 2
NS = 16
LANE = 128


def _mesh():
    return plsc.VectorSubcoreMesh(
        core_axis_name="c", subcore_axis_name="s", num_cores=NC, num_subcores=NS
    )


def _make_deg(n_pad, e_rows):
    rows_per_tile = e_rows // (NC * NS)
    n_per_tile = n_pad // NS

    @functools.partial(
        pl.kernel,
        out_type=(
            jax.ShapeDtypeStruct((n_pad,), F32),
            jax.ShapeDtypeStruct((n_pad,), F32),
        ),
        mesh=_mesh(),
        scratch_types=[
            pltpu.VMEM_SHARED((n_pad,), F32),
            pltpu.VMEM((rows_per_tile, LANE), jnp.int32),
            pltpu.VMEM((LANE,), F32),
            pltpu.VMEM((n_per_tile,), F32),
        ],
    )
    def deg_kernel(dst2d, zeros1, out0, out1, acc, idx_v, ones_v, wb_v):
        c = lax.axis_index("c")
        s = lax.axis_index("s")
        pltpu.sync_copy(
            zeros1.at[pl.ds(s * n_per_tile, n_per_tile)],
            acc.at[pl.ds(s * n_per_tile, n_per_tile)],
        )
        for i in range(LANE // 16):
            ones_v[pl.ds(i * 16, 16)] = jnp.full((16,), 1.0, F32)
        row0 = c * (e_rows // NC) + s * rows_per_tile
        pltpu.sync_copy(dst2d.at[pl.ds(row0, rows_per_tile)], idx_v)
        plsc.subcore_barrier()

        def body(j, _):
            pltpu.sync_copy(ones_v, acc.at[idx_v.at[j]], add=True)
            return 0

        lax.fori_loop(0, rows_per_tile, body, 0)
        plsc.subcore_barrier()
        sl = pl.ds(s * n_per_tile, n_per_tile)
        pltpu.sync_copy(acc.at[sl], wb_v)

        @pl.when(c == 0)
        def _():
            pltpu.sync_copy(wb_v, out0.at[sl])

        @pl.when(c == 1)
        def _():
            pltpu.sync_copy(wb_v, out1.at[sl])

    return deg_kernel


AGG_DT = jnp.float32


def _make_agg(n_pad, e_rows, half):
    base, rem = divmod(e_rows, NS)
    n_per_tile = n_pad // NS
    wb_chunks = n_per_tile // LANE

    nib = 4
    ngb = 2
    assert base >= nib

    @functools.partial(
        pl.kernel,
        out_type=(
            jax.ShapeDtypeStruct((n_pad, half), AGG_DT),
            jax.ShapeDtypeStruct((n_pad, half), AGG_DT),
        ),
        mesh=_mesh(),
        scratch_types=[
            pltpu.VMEM_SHARED((n_pad, half), AGG_DT),
            [pltpu.VMEM((2, LANE), jnp.int32) for _ in range(nib)],
            [pltpu.VMEM((LANE, half), AGG_DT) for _ in range(ngb)],
            [pltpu.SemaphoreType.DMA for _ in range(nib)],
            [pltpu.SemaphoreType.DMA for _ in range(ngb)],
            [pltpu.SemaphoreType.DMA for _ in range(2)],
        ],
    )
    def agg_kernel(edg3d, tab_lo, tab_hi, zeros2,
                   out_lo, out_hi, acc, idx_v, rows_v, isems, gsems, wsems):
        c = lax.axis_index("c")
        s = lax.axis_index("s")
        nsl = pl.ds(s * n_per_tile, n_per_tile)
        nch = base + jnp.where(s < rem, 1, 0)
        row0 = s * base + jnp.minimum(s, rem)

        def run(tab, out):
            def prefetch(j, ib):
                pltpu.async_copy(edg3d.at[row0 + j], idx_v[ib], isems[ib])

            def wait_idx(ib):
                pltpu.make_async_copy(edg3d.at[row0], idx_v[ib],
                                      isems[ib]).wait()

            def gather(ib, gb):
                pltpu.async_copy(tab.at[idx_v[ib].at[0]], rows_v[gb],
                                 gsems[gb])

            def wait_gather(gb):
                pltpu.make_async_copy(tab.at[idx_v[0].at[0]], rows_v[gb],
                                      gsems[gb]).wait()

            for j in range(nib):
                prefetch(j, j)
            pltpu.sync_copy(zeros2.at[nsl], acc.at[nsl])
            for j in range(ngb):
                wait_idx(j)
                gather(j, j)
            plsc.subcore_barrier()

            def outer(i, _):
                for b in range(nib):
                    j = i * nib + b
                    gb = b % ngb
                    ib2 = (b + ngb) % nib

                    def step(j=j, b=b, gb=gb, ib2=ib2):
                        wait_gather(gb)
                        pltpu.sync_copy(rows_v[gb], acc.at[idx_v[b].at[1]],
                                        add=True)
                        pl.when(j + nib < nch)(
                            lambda: prefetch(j + nib, b))

                        def nxt_gather():
                            wait_idx(ib2)
                            gather(ib2, gb)

                        pl.when(j + ngb < nch)(nxt_gather)

                    pl.when(j < nch)(step)
                return 0

            lax.fori_loop(0, (base + nib) // nib, outer, 0)
            plsc.subcore_barrier()
            for q in range(wb_chunks):
                u = q % 2
                sl = pl.ds(s * n_per_tile + q * LANE, LANE)
                if q >= 2:
                    pltpu.make_async_copy(rows_v[u], out.at[sl],
                                          wsems[u]).wait()
                pltpu.sync_copy(acc.at[sl], rows_v[u])
                pltpu.async_copy(rows_v[u], out.at[sl], wsems[u])
            last = pl.ds(s * n_per_tile, LANE)
            for u in range(2):
                pltpu.make_async_copy(rows_v[u], out.at[last],
                                      wsems[u]).wait()

        @pl.when(c == 0)
        def _():
            run(tab_lo, out_lo)

        @pl.when(c == 1)
        def _():
            run(tab_hi, out_hi)

    return agg_kernel


def _tc_scale(deg0, deg1, x_pad, half):
    n_pad, fin = x_pad.shape
    blk = 1024
    grid = (n_pad // blk,)

    def body(d0, d1, x, lo, hi):
        dinv = lax.rsqrt(d0[...] + d1[...] + 1.0)
        xs = (x[...] * dinv[:, None]).astype(AGG_DT)
        lo[...] = xs[:, :half]
        hi[...] = xs[:, half:]

    return pl.pallas_call(
        body,
        grid=grid,
        in_specs=[
            pl.BlockSpec((blk,), lambda i: (i,)),
            pl.BlockSpec((blk,), lambda i: (i,)),
            pl.BlockSpec((blk, fin), lambda i: (i, 0)),
        ],
        out_specs=[
            pl.BlockSpec((blk, half), lambda i: (i, 0)),
            pl.BlockSpec((blk, half), lambda i: (i, 0)),
        ],
        out_shape=[
            jax.ShapeDtypeStruct((n_pad, half), AGG_DT),
            jax.ShapeDtypeStruct((n_pad, half), AGG_DT),
        ],
    )(deg0, deg1, x_pad)


def _tc_layer1(deg0, deg1, x_pad, u_lo, u_hi, W1, b1, W2, half):
    n_pad, fin = x_pad.shape
    fmid = W1.shape[1]
    blk = 1024
    grid = (n_pad // blk,)

    def body(d0, d1, x, ulo, uhi, w1, bb1, w2, t_out, tslo, tshi):
        dinv = lax.rsqrt(d0[...] + d1[...] + 1.0)
        u = jnp.concatenate([ulo[...], uhi[...]], axis=1).astype(F32)
        z = u * dinv[:, None] + x[...] * (dinv * dinv)[:, None]
        bf = jnp.bfloat16
        h = jnp.maximum(
            jnp.dot(z.astype(bf), w1[...].astype(bf),
                    preferred_element_type=F32) + bb1[...][None, :],
            0.0,
        )
        t = jnp.dot(h.astype(bf), w2[...].astype(bf),
                    preferred_element_type=F32)
        t_out[...] = t
        ts = (t * dinv[:, None]).astype(AGG_DT)
        tslo[...] = ts[:, :half]
        tshi[...] = ts[:, half:]

    return pl.pallas_call(
        body,
        grid=grid,
        in_specs=[
            pl.BlockSpec((blk,), lambda i: (i,)),
            pl.BlockSpec((blk,), lambda i: (i,)),
            pl.BlockSpec((blk, fin), lambda i: (i, 0)),
            pl.BlockSpec((blk, half), lambda i: (i, 0)),
            pl.BlockSpec((blk, half), lambda i: (i, 0)),
            pl.BlockSpec((fin, fmid), lambda i: (0, 0)),
            pl.BlockSpec((fmid,), lambda i: (0,)),
            pl.BlockSpec((fmid, fin), lambda i: (0, 0)),
        ],
        out_specs=[
            pl.BlockSpec((blk, fin), lambda i: (i, 0)),
            pl.BlockSpec((blk, half), lambda i: (i, 0)),
            pl.BlockSpec((blk, half), lambda i: (i, 0)),
        ],
        out_shape=[
            jax.ShapeDtypeStruct((n_pad, fin), F32),
            jax.ShapeDtypeStruct((n_pad, half), AGG_DT),
            jax.ShapeDtypeStruct((n_pad, half), AGG_DT),
        ],
    )(deg0, deg1, x_pad, u_lo, u_hi, W1, b1, W2)


def _tc_finish(deg0, deg1, t, u_lo, u_hi, b2):
    n_pad, fout = t.shape
    half = fout // 2
    blk = 1024
    grid = (n_pad // blk,)

    def body(d0, d1, tt, ulo, uhi, bb2, out):
        dinv = lax.rsqrt(d0[...] + d1[...] + 1.0)
        u = jnp.concatenate([ulo[...], uhi[...]], axis=1).astype(F32)
        z = u * dinv[:, None] + tt[...] * (dinv * dinv)[:, None] + bb2[...][None, :]
        r = jnp.maximum(z, 0.0)
        m = jnp.max(r, axis=1, keepdims=True)
        lse = m + jnp.log(jnp.sum(jnp.exp(r - m), axis=1, keepdims=True))
        out[...] = r - lse

    return pl.pallas_call(
        body,
        grid=grid,
        in_specs=[
            pl.BlockSpec((blk,), lambda i: (i,)),
            pl.BlockSpec((blk,), lambda i: (i,)),
            pl.BlockSpec((blk, fout), lambda i: (i, 0)),
            pl.BlockSpec((blk, half), lambda i: (i, 0)),
            pl.BlockSpec((blk, half), lambda i: (i, 0)),
            pl.BlockSpec((fout,), lambda i: (0,)),
        ],
        out_specs=pl.BlockSpec((blk, fout), lambda i: (i, 0)),
        out_shape=jax.ShapeDtypeStruct((n_pad, fout), F32),
    )(deg0, deg1, t, u_lo, u_hi, b2)


def kernel(x, edge_index, W1, b1, W2, b2):
    n, fin = x.shape
    half = fin // 2
    e = edge_index.shape[1]

    n_pad = ((n + 1 + 1023) // 1024) * 1024
    e_pad_d = ((e + NC * NS * LANE - 1) // (NC * NS * LANE)) * (NC * NS * LANE)
    e_pad_a = ((e + LANE - 1) // LANE) * LANE

    ei = edge_index.astype(jnp.int32)
    dst2d_deg = jnp.concatenate(
        [ei[1], jnp.full((e_pad_d - e,), n, jnp.int32)]
    ).reshape(e_pad_d // LANE, LANE)
    pad = jnp.full((e_pad_a - e,), n, jnp.int32)
    src2d = jnp.concatenate([ei[0], pad]).reshape(e_pad_a // LANE, LANE)
    dst2d = jnp.concatenate([ei[1], pad]).reshape(e_pad_a // LANE, LANE)
    edg3d = jnp.stack([src2d, dst2d], axis=1)
    x_pad = jnp.pad(x, ((0, n_pad - n), (0, 0)))
    zeros1 = jnp.zeros((n_pad,), F32)
    zeros2 = jnp.zeros((n_pad, half), AGG_DT)

    deg0, deg1 = _make_deg(n_pad, e_pad_d // LANE)(dst2d_deg, zeros1)
    xs_lo, xs_hi = _tc_scale(deg0, deg1, x_pad, half)
    agg = _make_agg(n_pad, e_pad_a // LANE, half)
    u1_lo, u1_hi = agg(edg3d, xs_lo, xs_hi, zeros2)
    t, ts_lo, ts_hi = _tc_layer1(deg0, deg1, x_pad, u1_lo, u1_hi, W1, b1, W2, half)
    u2_lo, u2_hi = agg(edg3d, ts_lo, ts_hi, zeros2)
    o = _tc_finish(deg0, deg1, t, u2_lo, u2_hi, b2)
    return o[:n]


# speedup vs baseline: 2.4143x; 1.0001x over previous
import functools

import jax
import jax.numpy as jnp
from jax import lax
from jax.experimental import pallas as pl
from jax.experimental.pallas import tpu as pltpu
from jax.experimental.pallas import tpu_sc as plsc

F32 = jnp.float32

NC = 2
NS = 16
LANE = 128


def _mesh():
    return plsc.VectorSubcoreMesh(
        core_axis_name="c", subcore_axis_name="s", num_cores=NC, num_subcores=NS
    )


def _make_deg(n_pad, e_rows):
    base, rem = divmod(e_rows, NC * NS)
    n_per_tile = n_pad // NS

    @functools.partial(
        pl.kernel,
        out_type=(
            jax.ShapeDtypeStruct((n_pad,), F32),
            jax.ShapeDtypeStruct((n_pad,), F32),
        ),
        mesh=_mesh(),
        scratch_types=[
            pltpu.VMEM_SHARED((n_pad,), F32),
            pltpu.VMEM((base + 1, 1, LANE), jnp.int32),
            pltpu.VMEM((LANE,), F32),
            pltpu.VMEM((n_per_tile,), F32),
        ],
    )
    def deg_kernel(dst2d, zeros1, out0, out1, acc, idx_v, ones_v, wb_v):
        c = lax.axis_index("c")
        s = lax.axis_index("s")
        pltpu.sync_copy(
            zeros1.at[pl.ds(s * n_per_tile, n_per_tile)],
            acc.at[pl.ds(s * n_per_tile, n_per_tile)],
        )
        for i in range(LANE // 16):
            ones_v[pl.ds(i * 16, 16)] = jnp.full((16,), 1.0, F32)
        wid = c * NS + s
        nch = base + jnp.where(wid < rem, 1, 0)
        row0 = wid * base + jnp.minimum(wid, rem)
        pltpu.sync_copy(dst2d.at[pl.ds(row0, base + 1)], idx_v)
        plsc.subcore_barrier()

        def body(j, _):
            pltpu.sync_copy(ones_v, acc.at[idx_v.at[j, 0]], add=True)
            return 0

        lax.fori_loop(0, nch, body, 0)
        plsc.subcore_barrier()
        sl = pl.ds(s * n_per_tile, n_per_tile)
        pltpu.sync_copy(acc.at[sl], wb_v)

        @pl.when(c == 0)
        def _():
            pltpu.sync_copy(wb_v, out0.at[sl])

        @pl.when(c == 1)
        def _():
            pltpu.sync_copy(wb_v, out1.at[sl])

    return deg_kernel


AGG_DT = jnp.float32


def _make_agg(n_pad, e_rows, half):
    base, rem = divmod(e_rows, NS)
    n_per_tile = n_pad // NS
    wb_chunks = n_per_tile // LANE

    nib = 4
    ngb = 2
    assert base >= nib

    @functools.partial(
        pl.kernel,
        out_type=(
            jax.ShapeDtypeStruct((n_pad, half), AGG_DT),
            jax.ShapeDtypeStruct((n_pad, half), AGG_DT),
        ),
        mesh=_mesh(),
        scratch_types=[
            pltpu.VMEM_SHARED((n_pad, half), AGG_DT),
            [pltpu.VMEM((2, LANE), jnp.int32) for _ in range(nib)],
            [pltpu.VMEM((LANE, half), AGG_DT) for _ in range(ngb)],
            [pltpu.SemaphoreType.DMA for _ in range(nib)],
            [pltpu.SemaphoreType.DMA for _ in range(ngb)],
            [pltpu.SemaphoreType.DMA for _ in range(2)],
        ],
    )
    def agg_kernel(edg3d, tab_lo, tab_hi, zeros2,
                   out_lo, out_hi, acc, idx_v, rows_v, isems, gsems, wsems):
        c = lax.axis_index("c")
        s = lax.axis_index("s")
        nsl = pl.ds(s * n_per_tile, n_per_tile)
        nch = base + jnp.where(s < rem, 1, 0)
        row0 = s * base + jnp.minimum(s, rem)

        def run(tab, out):
            def prefetch(j, ib):
                pltpu.async_copy(edg3d.at[row0 + j], idx_v[ib], isems[ib])

            def wait_idx(ib):
                pltpu.make_async_copy(edg3d.at[row0], idx_v[ib],
                                      isems[ib]).wait()

            def gather(ib, gb):
                pltpu.async_copy(tab.at[idx_v[ib].at[0]], rows_v[gb],
                                 gsems[gb])

            def wait_gather(gb):
                pltpu.make_async_copy(tab.at[idx_v[0].at[0]], rows_v[gb],
                                      gsems[gb]).wait()

            for j in range(nib):
                prefetch(j, j)
            pltpu.sync_copy(zeros2.at[nsl], acc.at[nsl])
            for j in range(ngb):
                wait_idx(j)
                gather(j, j)
            plsc.subcore_barrier()

            def outer(i, _):
                for b in range(nib):
                    j = i * nib + b
                    gb = b % ngb
                    ib2 = (b + ngb) % nib

                    def step(j=j, b=b, gb=gb, ib2=ib2):
                        wait_gather(gb)
                        pltpu.sync_copy(rows_v[gb], acc.at[idx_v[b].at[1]],
                                        add=True)
                        pl.when(j + nib < nch)(
                            lambda: prefetch(j + nib, b))

                        def nxt_gather():
                            wait_idx(ib2)
                            gather(ib2, gb)

                        pl.when(j + ngb < nch)(nxt_gather)

                    pl.when(j < nch)(step)
                return 0

            lax.fori_loop(0, (base + nib) // nib, outer, 0)
            plsc.subcore_barrier()
            for q in range(wb_chunks):
                u = q % 2
                sl = pl.ds(s * n_per_tile + q * LANE, LANE)
                if q >= 2:
                    pltpu.make_async_copy(rows_v[u], out.at[sl],
                                          wsems[u]).wait()
                pltpu.sync_copy(acc.at[sl], rows_v[u])
                pltpu.async_copy(rows_v[u], out.at[sl], wsems[u])
            last = pl.ds(s * n_per_tile, LANE)
            for u in range(2):
                pltpu.make_async_copy(rows_v[u], out.at[last],
                                      wsems[u]).wait()

        @pl.when(c == 0)
        def _():
            run(tab_lo, out_lo)

        @pl.when(c == 1)
        def _():
            run(tab_hi, out_hi)

    return agg_kernel


def _tc_scale(deg0, deg1, x_pad, half):
    n_pad, fin = x_pad.shape
    blk = 1024
    grid = (n_pad // blk,)

    def body(d0, d1, x, lo, hi):
        dinv = lax.rsqrt(d0[...] + d1[...] + 1.0)
        xs = (x[...] * dinv[:, None]).astype(AGG_DT)
        lo[...] = xs[:, :half]
        hi[...] = xs[:, half:]

    return pl.pallas_call(
        body,
        grid=grid,
        in_specs=[
            pl.BlockSpec((blk,), lambda i: (i,)),
            pl.BlockSpec((blk,), lambda i: (i,)),
            pl.BlockSpec((blk, fin), lambda i: (i, 0)),
        ],
        out_specs=[
            pl.BlockSpec((blk, half), lambda i: (i, 0)),
            pl.BlockSpec((blk, half), lambda i: (i, 0)),
        ],
        out_shape=[
            jax.ShapeDtypeStruct((n_pad, half), AGG_DT),
            jax.ShapeDtypeStruct((n_pad, half), AGG_DT),
        ],
    )(deg0, deg1, x_pad)


def _tc_layer1(deg0, deg1, x_pad, u_lo, u_hi, W1, b1, W2, half):
    n_pad, fin = x_pad.shape
    fmid = W1.shape[1]
    blk = 1024
    grid = (n_pad // blk,)

    def body(d0, d1, x, ulo, uhi, w1, bb1, w2, t_out, tslo, tshi):
        dinv = lax.rsqrt(d0[...] + d1[...] + 1.0)
        u = jnp.concatenate([ulo[...], uhi[...]], axis=1).astype(F32)
        z = u * dinv[:, None] + x[...] * (dinv * dinv)[:, None]
        bf = jnp.bfloat16
        h = jnp.maximum(
            jnp.dot(z.astype(bf), w1[...].astype(bf),
                    preferred_element_type=F32) + bb1[...][None, :],
            0.0,
        )
        t = jnp.dot(h.astype(bf), w2[...].astype(bf),
                    preferred_element_type=F32)
        t_out[...] = t
        ts = (t * dinv[:, None]).astype(AGG_DT)
        tslo[...] = ts[:, :half]
        tshi[...] = ts[:, half:]

    return pl.pallas_call(
        body,
        grid=grid,
        in_specs=[
            pl.BlockSpec((blk,), lambda i: (i,)),
            pl.BlockSpec((blk,), lambda i: (i,)),
            pl.BlockSpec((blk, fin), lambda i: (i, 0)),
            pl.BlockSpec((blk, half), lambda i: (i, 0)),
            pl.BlockSpec((blk, half), lambda i: (i, 0)),
            pl.BlockSpec((fin, fmid), lambda i: (0, 0)),
            pl.BlockSpec((fmid,), lambda i: (0,)),
            pl.BlockSpec((fmid, fin), lambda i: (0, 0)),
        ],
        out_specs=[
            pl.BlockSpec((blk, fin), lambda i: (i, 0)),
            pl.BlockSpec((blk, half), lambda i: (i, 0)),
            pl.BlockSpec((blk, half), lambda i: (i, 0)),
        ],
        out_shape=[
            jax.ShapeDtypeStruct((n_pad, fin), F32),
            jax.ShapeDtypeStruct((n_pad, half), AGG_DT),
            jax.ShapeDtypeStruct((n_pad, half), AGG_DT),
        ],
    )(deg0, deg1, x_pad, u_lo, u_hi, W1, b1, W2)


def _tc_finish(deg0, deg1, t, u_lo, u_hi, b2):
    n_pad, fout = t.shape
    half = fout // 2
    blk = 1024
    grid = (n_pad // blk,)

    def body(d0, d1, tt, ulo, uhi, bb2, out):
        dinv = lax.rsqrt(d0[...] + d1[...] + 1.0)
        u = jnp.concatenate([ulo[...], uhi[...]], axis=1).astype(F32)
        z = u * dinv[:, None] + tt[...] * (dinv * dinv)[:, None] + bb2[...][None, :]
        r = jnp.maximum(z, 0.0)
        m = jnp.max(r, axis=1, keepdims=True)
        lse = m + jnp.log(jnp.sum(jnp.exp(r - m), axis=1, keepdims=True))
        out[...] = r - lse

    return pl.pallas_call(
        body,
        grid=grid,
        in_specs=[
            pl.BlockSpec((blk,), lambda i: (i,)),
            pl.BlockSpec((blk,), lambda i: (i,)),
            pl.BlockSpec((blk, fout), lambda i: (i, 0)),
            pl.BlockSpec((blk, half), lambda i: (i, 0)),
            pl.BlockSpec((blk, half), lambda i: (i, 0)),
            pl.BlockSpec((fout,), lambda i: (0,)),
        ],
        out_specs=pl.BlockSpec((blk, fout), lambda i: (i, 0)),
        out_shape=jax.ShapeDtypeStruct((n_pad, fout), F32),
    )(deg0, deg1, t, u_lo, u_hi, b2)


def kernel(x, edge_index, W1, b1, W2, b2):
    n, fin = x.shape
    half = fin // 2
    e = edge_index.shape[1]

    n_pad = ((n + 1 + 1023) // 1024) * 1024
    e_pad_a = ((e + LANE - 1) // LANE) * LANE

    ei = edge_index.astype(jnp.int32)
    dst2d_deg = jnp.concatenate(
        [ei[1], jnp.full((e_pad_a - e + LANE,), n, jnp.int32)]
    ).reshape(e_pad_a // LANE + 1, 1, LANE)
    pad = jnp.full((e_pad_a - e,), n, jnp.int32)
    src2d = jnp.concatenate([ei[0], pad]).reshape(e_pad_a // LANE, LANE)
    dst2d = jnp.concatenate([ei[1], pad]).reshape(e_pad_a // LANE, LANE)
    edg3d = jnp.stack([src2d, dst2d], axis=1)
    x_pad = jnp.pad(x, ((0, n_pad - n), (0, 0)))
    zeros1 = jnp.zeros((n_pad,), F32)
    zeros2 = jnp.zeros((n_pad, half), AGG_DT)

    deg0, deg1 = _make_deg(n_pad, e_pad_a // LANE)(dst2d_deg, zeros1)
    xs_lo, xs_hi = _tc_scale(deg0, deg1, x_pad, half)
    agg = _make_agg(n_pad, e_pad_a // LANE, half)
    u1_lo, u1_hi = agg(edg3d, xs_lo, xs_hi, zeros2)
    t, ts_lo, ts_hi = _tc_layer1(deg0, deg1, x_pad, u1_lo, u1_hi, W1, b1, W2, half)
    u2_lo, u2_hi = agg(edg3d, ts_lo, ts_hi, zeros2)
    o = _tc_finish(deg0, deg1, t, u2_lo, u2_hi, b2)
    return o[:n]


# speedup vs baseline: 2.5083x; 1.0389x over previous
import functools

import jax
import jax.numpy as jnp
from jax import lax
from jax.experimental import pallas as pl
from jax.experimental.pallas import tpu as pltpu
from jax.experimental.pallas import tpu_sc as plsc

F32 = jnp.float32

NC = 2
NS = 16
LANE = 128


def _mesh():
    return plsc.VectorSubcoreMesh(
        core_axis_name="c", subcore_axis_name="s", num_cores=NC, num_subcores=NS
    )


def _make_deg(n_pad, e_rows):
    base, rem = divmod(e_rows, NC * NS)
    n_per_tile = n_pad // NS

    @functools.partial(
        pl.kernel,
        out_type=(
            jax.ShapeDtypeStruct((n_pad,), F32),
            jax.ShapeDtypeStruct((n_pad,), F32),
        ),
        mesh=_mesh(),
        scratch_types=[
            pltpu.VMEM_SHARED((n_pad,), F32),
            pltpu.VMEM((base + 1, 1, LANE), jnp.int32),
            pltpu.VMEM((LANE,), F32),
            pltpu.VMEM((n_per_tile,), F32),
        ],
    )
    def deg_kernel(dst2d, zeros1, out0, out1, acc, idx_v, ones_v, wb_v):
        c = lax.axis_index("c")
        s = lax.axis_index("s")
        pltpu.sync_copy(
            zeros1.at[pl.ds(s * n_per_tile, n_per_tile)],
            acc.at[pl.ds(s * n_per_tile, n_per_tile)],
        )
        for i in range(LANE // 16):
            ones_v[pl.ds(i * 16, 16)] = jnp.full((16,), 1.0, F32)
        wid = c * NS + s
        nch = base + jnp.where(wid < rem, 1, 0)
        row0 = wid * base + jnp.minimum(wid, rem)
        pltpu.sync_copy(dst2d.at[pl.ds(row0, base + 1)], idx_v)
        plsc.subcore_barrier()

        def body(j, _):
            pltpu.sync_copy(ones_v, acc.at[idx_v.at[j, 0]], add=True)
            return 0

        lax.fori_loop(0, nch, body, 0)
        plsc.subcore_barrier()
        sl = pl.ds(s * n_per_tile, n_per_tile)
        pltpu.sync_copy(acc.at[sl], wb_v)

        @pl.when(c == 0)
        def _():
            pltpu.sync_copy(wb_v, out0.at[sl])

        @pl.when(c == 1)
        def _():
            pltpu.sync_copy(wb_v, out1.at[sl])

    return deg_kernel


AGG_DT = jnp.float32


CHKA = 96


def _make_agg(n_pad, e_rows, half):
    base, rem = divmod(e_rows, NS)
    n_per_tile = n_pad // NS
    wbc = 80
    wb_chunks = n_per_tile // wbc

    nib = 6
    ngb = 3
    assert base >= nib and n_per_tile % wbc == 0 and wbc <= CHKA

    @functools.partial(
        pl.kernel,
        out_type=(
            jax.ShapeDtypeStruct((n_pad, half), AGG_DT),
            jax.ShapeDtypeStruct((n_pad, half), AGG_DT),
        ),
        mesh=_mesh(),
        scratch_types=[
            pltpu.VMEM_SHARED((n_pad, half), AGG_DT),
            [pltpu.VMEM((2, CHKA), jnp.int32) for _ in range(nib)],
            [pltpu.VMEM((CHKA, half), AGG_DT) for _ in range(ngb)],
            [pltpu.SemaphoreType.DMA for _ in range(nib)],
            [pltpu.SemaphoreType.DMA for _ in range(ngb)],
            [pltpu.SemaphoreType.DMA for _ in range(ngb)],
            [pltpu.SemaphoreType.DMA for _ in range(2)],
        ],
    )
    def agg_kernel(edg3d, tab_lo, tab_hi, zeros2,
                   out_lo, out_hi, acc, idx_v, rows_v, isems, gsems,
                   scsems, wsems):
        c = lax.axis_index("c")
        s = lax.axis_index("s")
        nsl = pl.ds(s * n_per_tile, n_per_tile)
        nch = base + jnp.where(s < rem, 1, 0)
        row0 = s * base + jnp.minimum(s, rem)

        def run(tab, out):
            def prefetch(j, ib):
                pltpu.async_copy(edg3d.at[row0 + j], idx_v[ib], isems[ib])

            def wait_idx(ib):
                pltpu.make_async_copy(edg3d.at[row0], idx_v[ib],
                                      isems[ib]).wait()

            def gather(ib, gb):
                pltpu.async_copy(tab.at[idx_v[ib].at[0]], rows_v[gb],
                                 gsems[gb])

            def wait_gather(gb):
                pltpu.make_async_copy(tab.at[idx_v[0].at[0]], rows_v[gb],
                                      gsems[gb]).wait()

            def scatter(ib, gb):
                pltpu.async_copy(rows_v[gb], acc.at[idx_v[ib].at[1]],
                                 scsems[gb], add=True)

            def wait_scatter(gb):
                pltpu.make_async_copy(rows_v[gb], acc.at[idx_v[0].at[1]],
                                      scsems[gb]).wait()

            for j in range(4):
                prefetch(j, j)
            pltpu.sync_copy(zeros2.at[nsl], acc.at[nsl])
            for j in range(2):
                wait_idx(j)
                gather(j, j)
            plsc.subcore_barrier()

            def outer(i, _):
                for b in range(nib):
                    j = i * nib + b
                    gb = b % ngb

                    def step(j=j, b=b, gb=gb):
                        wait_gather(gb)
                        scatter(b, gb)
                        pl.when(j + 4 < nch)(
                            lambda: prefetch(j + 4, (b + 4) % nib))

                        def nxt_gather():
                            wait_idx((b + 2) % nib)
                            pl.when(j >= 1)(
                                lambda: wait_scatter((gb + 2) % ngb))
                            gather((b + 2) % nib, (gb + 2) % ngb)

                        pl.when(j + 2 < nch)(nxt_gather)

                    pl.when(j < nch)(step)
                return 0

            lax.fori_loop(0, (base + nib) // nib, outer, 0)
            for u in range(ngb):
                wait_scatter(u)
            plsc.subcore_barrier()
            for q in range(wb_chunks):
                u = q % 2
                sl = pl.ds(s * n_per_tile + q * wbc, wbc)
                buf = rows_v[u].at[pl.ds(0, wbc)]
                if q >= 2:
                    pltpu.make_async_copy(buf, out.at[sl], wsems[u]).wait()
                pltpu.sync_copy(acc.at[sl], buf)
                pltpu.async_copy(buf, out.at[sl], wsems[u])
            last = pl.ds(s * n_per_tile, wbc)
            for u in range(2):
                pltpu.make_async_copy(rows_v[u].at[pl.ds(0, wbc)],
                                      out.at[last], wsems[u]).wait()

        @pl.when(c == 0)
        def _():
            run(tab_lo, out_lo)

        @pl.when(c == 1)
        def _():
            run(tab_hi, out_hi)

    return agg_kernel


def _tc_scale(deg0, deg1, x_pad, half):
    n_pad, fin = x_pad.shape
    blk = 1024
    grid = (n_pad // blk,)

    def body(d0, d1, x, lo, hi):
        dinv = lax.rsqrt(d0[...] + d1[...] + 1.0)
        xs = (x[...] * dinv[:, None]).astype(AGG_DT)
        lo[...] = xs[:, :half]
        hi[...] = xs[:, half:]

    return pl.pallas_call(
        body,
        grid=grid,
        in_specs=[
            pl.BlockSpec((blk,), lambda i: (i,)),
            pl.BlockSpec((blk,), lambda i: (i,)),
            pl.BlockSpec((blk, fin), lambda i: (i, 0)),
        ],
        out_specs=[
            pl.BlockSpec((blk, half), lambda i: (i, 0)),
            pl.BlockSpec((blk, half), lambda i: (i, 0)),
        ],
        out_shape=[
            jax.ShapeDtypeStruct((n_pad, half), AGG_DT),
            jax.ShapeDtypeStruct((n_pad, half), AGG_DT),
        ],
    )(deg0, deg1, x_pad)


def _tc_layer1(deg0, deg1, x_pad, u_lo, u_hi, W1, b1, W2, half):
    n_pad, fin = x_pad.shape
    fmid = W1.shape[1]
    blk = 1024
    grid = (n_pad // blk,)

    def body(d0, d1, x, ulo, uhi, w1, bb1, w2, t_out, tslo, tshi):
        dinv = lax.rsqrt(d0[...] + d1[...] + 1.0)
        u = jnp.concatenate([ulo[...], uhi[...]], axis=1).astype(F32)
        z = u * dinv[:, None] + x[...] * (dinv * dinv)[:, None]
        bf = jnp.bfloat16
        h = jnp.maximum(
            jnp.dot(z.astype(bf), w1[...].astype(bf),
                    preferred_element_type=F32) + bb1[...][None, :],
            0.0,
        )
        t = jnp.dot(h.astype(bf), w2[...].astype(bf),
                    preferred_element_type=F32)
        t_out[...] = t
        ts = (t * dinv[:, None]).astype(AGG_DT)
        tslo[...] = ts[:, :half]
        tshi[...] = ts[:, half:]

    return pl.pallas_call(
        body,
        grid=grid,
        in_specs=[
            pl.BlockSpec((blk,), lambda i: (i,)),
            pl.BlockSpec((blk,), lambda i: (i,)),
            pl.BlockSpec((blk, fin), lambda i: (i, 0)),
            pl.BlockSpec((blk, half), lambda i: (i, 0)),
            pl.BlockSpec((blk, half), lambda i: (i, 0)),
            pl.BlockSpec((fin, fmid), lambda i: (0, 0)),
            pl.BlockSpec((fmid,), lambda i: (0,)),
            pl.BlockSpec((fmid, fin), lambda i: (0, 0)),
        ],
        out_specs=[
            pl.BlockSpec((blk, fin), lambda i: (i, 0)),
            pl.BlockSpec((blk, half), lambda i: (i, 0)),
            pl.BlockSpec((blk, half), lambda i: (i, 0)),
        ],
        out_shape=[
            jax.ShapeDtypeStruct((n_pad, fin), F32),
            jax.ShapeDtypeStruct((n_pad, half), AGG_DT),
            jax.ShapeDtypeStruct((n_pad, half), AGG_DT),
        ],
    )(deg0, deg1, x_pad, u_lo, u_hi, W1, b1, W2)


def _tc_finish(deg0, deg1, t, u_lo, u_hi, b2):
    n_pad, fout = t.shape
    half = fout // 2
    blk = 1024
    grid = (n_pad // blk,)

    def body(d0, d1, tt, ulo, uhi, bb2, out):
        dinv = lax.rsqrt(d0[...] + d1[...] + 1.0)
        u = jnp.concatenate([ulo[...], uhi[...]], axis=1).astype(F32)
        z = u * dinv[:, None] + tt[...] * (dinv * dinv)[:, None] + bb2[...][None, :]
        r = jnp.maximum(z, 0.0)
        m = jnp.max(r, axis=1, keepdims=True)
        lse = m + jnp.log(jnp.sum(jnp.exp(r - m), axis=1, keepdims=True))
        out[...] = r - lse

    return pl.pallas_call(
        body,
        grid=grid,
        in_specs=[
            pl.BlockSpec((blk,), lambda i: (i,)),
            pl.BlockSpec((blk,), lambda i: (i,)),
            pl.BlockSpec((blk, fout), lambda i: (i, 0)),
            pl.BlockSpec((blk, half), lambda i: (i, 0)),
            pl.BlockSpec((blk, half), lambda i: (i, 0)),
            pl.BlockSpec((fout,), lambda i: (0,)),
        ],
        out_specs=pl.BlockSpec((blk, fout), lambda i: (i, 0)),
        out_shape=jax.ShapeDtypeStruct((n_pad, fout), F32),
    )(deg0, deg1, t, u_lo, u_hi, b2)


def kernel(x, edge_index, W1, b1, W2, b2):
    n, fin = x.shape
    half = fin // 2
    e = edge_index.shape[1]

    n_pad = ((n + 1 + 1023) // 1024) * 1024
    e_pad_d = ((e + LANE - 1) // LANE) * LANE
    e_pad_a = ((e + CHKA - 1) // CHKA) * CHKA

    ei = edge_index.astype(jnp.int32)
    dst2d_deg = jnp.concatenate(
        [ei[1], jnp.full((e_pad_d - e + LANE,), n, jnp.int32)]
    ).reshape(e_pad_d // LANE + 1, 1, LANE)
    pad = jnp.full((e_pad_a - e,), n, jnp.int32)
    src2d = jnp.concatenate([ei[0], pad]).reshape(e_pad_a // CHKA, CHKA)
    dst2d = jnp.concatenate([ei[1], pad]).reshape(e_pad_a // CHKA, CHKA)
    edg3d = jnp.stack([src2d, dst2d], axis=1)
    x_pad = jnp.pad(x, ((0, n_pad - n), (0, 0)))
    zeros1 = jnp.zeros((n_pad,), F32)
    zeros2 = jnp.zeros((n_pad, half), AGG_DT)

    deg0, deg1 = _make_deg(n_pad, e_pad_d // LANE)(dst2d_deg, zeros1)
    xs_lo, xs_hi = _tc_scale(deg0, deg1, x_pad, half)
    agg = _make_agg(n_pad, e_pad_a // CHKA, half)
    u1_lo, u1_hi = agg(edg3d, xs_lo, xs_hi, zeros2)
    t, ts_lo, ts_hi = _tc_layer1(deg0, deg1, x_pad, u1_lo, u1_hi, W1, b1, W2, half)
    u2_lo, u2_hi = agg(edg3d, ts_lo, ts_hi, zeros2)
    o = _tc_finish(deg0, deg1, t, u2_lo, u2_hi, b2)
    return o[:n]
